# baseline (speedup 1.0000x reference)
import functools

import jax
import jax.numpy as jnp
from jax import lax
from jax.experimental import pallas as pl
from jax.experimental.pallas import tpu as pltpu
from jax.experimental.pallas import tpu_sc as plsc

_BM = 128
_SUB = 32
_LANE = 128


def _masked_shift(v, s, axis):
    rolled = jnp.roll(v, s, axis=axis)
    idx = jax.lax.broadcasted_iota(jnp.int32, v.shape, axis)
    return jnp.where(idx >= s, rolled, 0)


def _cumsum2d(m):
    s = 1
    while s < _LANE:
        m = m + _masked_shift(m, s, 1)
        s *= 2
    row_tot = jax.lax.broadcast_in_dim(m[:, _LANE - 1], (_SUB, 1), (0,))
    row_tot = jnp.broadcast_to(row_tot, (_SUB, _LANE))
    carry = _masked_shift(row_tot, 1, 0)
    s = 1
    while s < _SUB:
        carry = carry + _masked_shift(carry, s, 0)
        s *= 2
    return m + carry


_NBUF = 6
_LOOKAHEAD = 5


def _routing_kernel(ids_ref, dest_ref, bexp_ref, first_ref, slot_ref, E: int):
    ids = ids_ref[...]
    dest = jnp.zeros((_SUB, _LANE), jnp.int32)
    bexp = jnp.zeros((1, _LANE), jnp.int32)
    blk_iota = jax.lax.broadcasted_iota(jnp.int32, (1, _LANE), 1) * _BM
    padded_start = jnp.int32(0)
    for e in range(E):
        m = (ids == e).astype(jnp.int32)
        csum = _cumsum2d(m)
        count = csum[_SUB - 1, _LANE - 1]
        dest = dest + m * (padded_start + csum - 1)
        padded_start = padded_start + ((count + _BM - 1) // _BM) * _BM
        if e < E - 1:
            bexp = bexp + (blk_iota >= padded_start).astype(jnp.int32)
    bexp = jnp.minimum(bexp, E - 1)
    lane = jax.lax.broadcasted_iota(jnp.int32, (1, _LANE), 1)
    first = (bexp != _masked_shift(bexp, 1, 1)).astype(jnp.int32)
    first = jnp.where(lane == 0, 1, first)
    runidx = first
    s = 1
    while s < _LANE:
        runidx = runidx + _masked_shift(runidx, s, 1)
        s *= 2
    runidx = runidx - 1
    dest_ref[...] = dest
    bexp_ref[...] = bexp
    first_ref[...] = first
    slot_ref[...] = runidx % _NBUF


def _gemm_silu_kernel(be_ref, first_ref, slot_ref, x_ref, w_hbm, o_ref, wbuf, sems):
    i = pl.program_id(0)
    nb = pl.num_programs(0)

    def _issue(j):
        @pl.when(jnp.logical_and(j < nb, first_ref[j] == 1))
        def _():
            for h in range(2):
                pltpu.make_async_copy(
                    w_hbm.at[be_ref[j], h],
                    wbuf.at[slot_ref[j], h],
                    sems.at[slot_ref[j], h],
                ).start()

    @pl.when(i == 0)
    def _():
        for j in range(_LOOKAHEAD):
            _issue(jnp.int32(j))

    _issue(i + _LOOKAHEAD)

    @pl.when(first_ref[i] == 1)
    def _():
        for h in range(2):
            pltpu.make_async_copy(
                w_hbm.at[be_ref[i], h],
                wbuf.at[slot_ref[i], h],
                sems.at[slot_ref[i], h],
            ).wait()

    s = slot_ref[i]
    a = x_ref[...]
    g = jax.lax.dot_general(
        a, wbuf[s, 0], (((1,), (1,)), ((), ())), preferred_element_type=jnp.float32
    )
    u = jax.lax.dot_general(
        a, wbuf[s, 1], (((1,), (1,)), ((), ())), preferred_element_type=jnp.float32
    )
    o_ref[...] = g * jax.nn.sigmoid(g) * u


def kernel(local_hidden_states, up_weight, full_topk_ids):
    T, K = local_hidden_states.shape
    E, N, _ = up_weight.shape
    topk = full_topk_ids.shape[1]
    M = T * topk
    N2 = N // 2

    ids2d = full_topk_ids.reshape(_SUB, _LANE).astype(jnp.int32)

    M_pad = M + E * _BM
    num_blocks = M_pad // _BM

    dest2d, bexp, first, slot = pl.pallas_call(
        functools.partial(_routing_kernel, E=E),
        out_shape=(
            jax.ShapeDtypeStruct((_SUB, _LANE), jnp.int32),
            jax.ShapeDtypeStruct((1, _LANE), jnp.int32),
            jax.ShapeDtypeStruct((1, _LANE), jnp.int32),
            jax.ShapeDtypeStruct((1, _LANE), jnp.int32),
        ),
    )(ids2d)
    dest = dest2d.reshape(M)

    dest_t = dest.reshape(T, topk).T
    info = plsc.get_sparse_core_info()
    nw = info.num_cores * info.num_subcores
    t_per_w = T // nw

    @functools.partial(
        pl.kernel,
        mesh=plsc.VectorSubcoreMesh(core_axis_name="c", subcore_axis_name="s"),
        out_type=jax.ShapeDtypeStruct((M_pad, K), jnp.float32),
        scratch_types=[
            pltpu.VMEM((t_per_w,), jnp.int32),
            pltpu.VMEM((t_per_w,), jnp.int32),
            pltpu.VMEM((t_per_w, K), jnp.float32),
            pltpu.SemaphoreType.DMA,
            pltpu.SemaphoreType.DMA,
        ],
    )
    def _dispatch(x_hbm, dest_hbm, out_hbm, idx0, idx1, xv, sem0, sem1):
        wid = lax.axis_index("s") * info.num_cores + lax.axis_index("c")
        base = wid * t_per_w
        pltpu.sync_copy(dest_hbm.at[0, pl.ds(base, t_per_w)], idx0)
        pltpu.sync_copy(dest_hbm.at[1, pl.ds(base, t_per_w)], idx1)
        pltpu.sync_copy(x_hbm.at[pl.ds(base, t_per_w)], xv)
        c0 = pltpu.async_copy(xv, out_hbm.at[idx0], sem0)
        c1 = pltpu.async_copy(xv, out_hbm.at[idx1], sem1)
        c0.wait()
        c1.wait()

    x_sorted = _dispatch(local_hidden_states, dest_t)

    w4 = up_weight.reshape(E, 2, N2, K)

    grid_spec = pltpu.PrefetchScalarGridSpec(
        num_scalar_prefetch=3,
        grid=(num_blocks,),
        in_specs=[
            pl.BlockSpec((_BM, K), lambda i, be, fi, sl: (i, 0)),
            pl.BlockSpec(memory_space=pl.ANY),
        ],
        out_specs=pl.BlockSpec((_BM, N2), lambda i, be, fi, sl: (i, 0)),
        scratch_shapes=[
            pltpu.VMEM((_NBUF, 2, N2, K), jnp.float32),
            pltpu.SemaphoreType.DMA((_NBUF, 2)),
        ],
    )
    out_sorted = pl.pallas_call(
        _gemm_silu_kernel,
        grid_spec=grid_spec,
        out_shape=jax.ShapeDtypeStruct((M_pad, N2), jnp.float32),
        compiler_params=pltpu.CompilerParams(
            dimension_semantics=("arbitrary",),
        ),
    )(bexp[0], first[0], slot[0], x_sorted, w4)

    return out_sorted[dest]


# speedup vs baseline: 1.2628x; 1.2628x over previous
import functools

import jax
import jax.numpy as jnp
from jax import lax
from jax.experimental import pallas as pl
from jax.experimental.pallas import tpu as pltpu
from jax.experimental.pallas import tpu_sc as plsc

_BM = 256
_SUB = 32
_LANE = 128


def _masked_shift(v, s, axis):
    rolled = jnp.roll(v, s, axis=axis)
    idx = jax.lax.broadcasted_iota(jnp.int32, v.shape, axis)
    return jnp.where(idx >= s, rolled, 0)


def _cumsum2d(m):
    s = 1
    while s < _LANE:
        m = m + _masked_shift(m, s, 1)
        s *= 2
    row_tot = jax.lax.broadcast_in_dim(m[:, _LANE - 1], (_SUB, 1), (0,))
    row_tot = jnp.broadcast_to(row_tot, (_SUB, _LANE))
    carry = _masked_shift(row_tot, 1, 0)
    s = 1
    while s < _SUB:
        carry = carry + _masked_shift(carry, s, 0)
        s *= 2
    return m + carry


_NBUF = 6
_LOOKAHEAD = 5


def _routing_kernel(ids_ref, dest_ref, bexp_ref, first_ref, slot_ref, E: int):
    ids = ids_ref[...]
    dest = jnp.zeros((_SUB, _LANE), jnp.int32)
    bexp = jnp.zeros((1, _LANE), jnp.int32)
    blk_iota = jax.lax.broadcasted_iota(jnp.int32, (1, _LANE), 1) * _BM
    padded_start = jnp.int32(0)
    for e in range(E):
        m = (ids == e).astype(jnp.int32)
        csum = _cumsum2d(m)
        count = csum[_SUB - 1, _LANE - 1]
        dest = dest + m * (padded_start + csum - 1)
        padded_start = padded_start + ((count + _BM - 1) // _BM) * _BM
        if e < E - 1:
            bexp = bexp + (blk_iota >= padded_start).astype(jnp.int32)
    bexp = jnp.minimum(bexp, E - 1)
    lane = jax.lax.broadcasted_iota(jnp.int32, (1, _LANE), 1)
    first = (bexp != _masked_shift(bexp, 1, 1)).astype(jnp.int32)
    first = jnp.where(lane == 0, 1, first)
    runidx = first
    s = 1
    while s < _LANE:
        runidx = runidx + _masked_shift(runidx, s, 1)
        s *= 2
    runidx = runidx - 1
    dest_ref[...] = dest
    bexp_ref[...] = bexp
    first_ref[...] = first
    slot_ref[...] = runidx % _NBUF


def _gemm_silu_kernel(be_ref, first_ref, slot_ref, x_ref, w_hbm, o_ref, wbuf, sems):
    i = pl.program_id(0)
    nb = pl.num_programs(0)

    def _issue(j):
        @pl.when(jnp.logical_and(j < nb, first_ref[j] == 1))
        def _():
            for h in range(2):
                pltpu.make_async_copy(
                    w_hbm.at[be_ref[j], h],
                    wbuf.at[slot_ref[j], h],
                    sems.at[slot_ref[j], h],
                ).start()

    @pl.when(i == 0)
    def _():
        for j in range(_LOOKAHEAD):
            _issue(jnp.int32(j))

    _issue(i + _LOOKAHEAD)

    @pl.when(first_ref[i] == 1)
    def _():
        for h in range(2):
            pltpu.make_async_copy(
                w_hbm.at[be_ref[i], h],
                wbuf.at[slot_ref[i], h],
                sems.at[slot_ref[i], h],
            ).wait()

    s = slot_ref[i]
    a = x_ref[...]
    g = jax.lax.dot_general(
        a, wbuf[s, 0], (((1,), (1,)), ((), ())), preferred_element_type=jnp.float32
    )
    u = jax.lax.dot_general(
        a, wbuf[s, 1], (((1,), (1,)), ((), ())), preferred_element_type=jnp.float32
    )
    o_ref[...] = g * jax.nn.sigmoid(g) * u


def kernel(local_hidden_states, up_weight, full_topk_ids):
    T, K = local_hidden_states.shape
    E, N, _ = up_weight.shape
    topk = full_topk_ids.shape[1]
    M = T * topk
    N2 = N // 2

    ids2d = full_topk_ids.reshape(_SUB, _LANE).astype(jnp.int32)

    M_pad = M + E * _BM
    num_blocks = M_pad // _BM

    dest2d, bexp, first, slot = pl.pallas_call(
        functools.partial(_routing_kernel, E=E),
        out_shape=(
            jax.ShapeDtypeStruct((_SUB, _LANE), jnp.int32),
            jax.ShapeDtypeStruct((1, _LANE), jnp.int32),
            jax.ShapeDtypeStruct((1, _LANE), jnp.int32),
            jax.ShapeDtypeStruct((1, _LANE), jnp.int32),
        ),
    )(ids2d)
    dest = dest2d.reshape(M)

    dest_t = dest.reshape(T, topk).T
    info = plsc.get_sparse_core_info()
    nw = info.num_cores * info.num_subcores
    t_per_w = T // nw

    @functools.partial(
        pl.kernel,
        mesh=plsc.VectorSubcoreMesh(core_axis_name="c", subcore_axis_name="s"),
        out_type=jax.ShapeDtypeStruct((M_pad, K), jnp.float32),
        scratch_types=[
            pltpu.VMEM((t_per_w,), jnp.int32),
            pltpu.VMEM((t_per_w,), jnp.int32),
            pltpu.VMEM((t_per_w, K), jnp.float32),
            pltpu.SemaphoreType.DMA,
            pltpu.SemaphoreType.DMA,
        ],
    )
    def _dispatch(x_hbm, dest_hbm, out_hbm, idx0, idx1, xv, sem0, sem1):
        wid = lax.axis_index("s") * info.num_cores + lax.axis_index("c")
        base = wid * t_per_w
        pltpu.sync_copy(dest_hbm.at[0, pl.ds(base, t_per_w)], idx0)
        pltpu.sync_copy(dest_hbm.at[1, pl.ds(base, t_per_w)], idx1)
        pltpu.sync_copy(x_hbm.at[pl.ds(base, t_per_w)], xv)
        c0 = pltpu.async_copy(xv, out_hbm.at[idx0], sem0)
        c1 = pltpu.async_copy(xv, out_hbm.at[idx1], sem1)
        c0.wait()
        c1.wait()

    x_sorted = _dispatch(local_hidden_states, dest_t)

    w4 = up_weight.reshape(E, 2, N2, K)

    grid_spec = pltpu.PrefetchScalarGridSpec(
        num_scalar_prefetch=3,
        grid=(num_blocks,),
        in_specs=[
            pl.BlockSpec((_BM, K), lambda i, be, fi, sl: (i, 0)),
            pl.BlockSpec(memory_space=pl.ANY),
        ],
        out_specs=pl.BlockSpec((_BM, N2), lambda i, be, fi, sl: (i, 0)),
        scratch_shapes=[
            pltpu.VMEM((_NBUF, 2, N2, K), jnp.float32),
            pltpu.SemaphoreType.DMA((_NBUF, 2)),
        ],
    )
    out_sorted = pl.pallas_call(
        _gemm_silu_kernel,
        grid_spec=grid_spec,
        out_shape=jax.ShapeDtypeStruct((M_pad, N2), jnp.float32),
        compiler_params=pltpu.CompilerParams(
            dimension_semantics=("arbitrary",),
        ),
    )(bexp[0], first[0], slot[0], x_sorted, w4)

    return out_sorted[dest]


# speedup vs baseline: 1.3006x; 1.0300x over previous
import functools

import jax
import jax.numpy as jnp
from jax import lax
from jax.experimental import pallas as pl
from jax.experimental.pallas import tpu as pltpu
from jax.experimental.pallas import tpu_sc as plsc

_BM = 256
_SUB = 32
_LANE = 128


def _masked_shift(v, s, axis):
    rolled = jnp.roll(v, s, axis=axis)
    idx = jax.lax.broadcasted_iota(jnp.int32, v.shape, axis)
    return jnp.where(idx >= s, rolled, 0)


def _cumsum2d(m):
    s = 1
    while s < _LANE:
        m = m + _masked_shift(m, s, 1)
        s *= 2
    row_tot = jax.lax.broadcast_in_dim(m[:, _LANE - 1], (_SUB, 1), (0,))
    row_tot = jnp.broadcast_to(row_tot, (_SUB, _LANE))
    carry = _masked_shift(row_tot, 1, 0)
    s = 1
    while s < _SUB:
        carry = carry + _masked_shift(carry, s, 0)
        s *= 2
    return m + carry


_NBUF = 6
_LOOKAHEAD = 5


def _routing_kernel(ids_ref, dest_ref, bexp_ref, first_ref, slot_ref, valid_ref, E: int):
    ids = ids_ref[...]
    dest = jnp.zeros((_SUB, _LANE), jnp.int32)
    bexp = jnp.zeros((1, _LANE), jnp.int32)
    blk_iota = jax.lax.broadcasted_iota(jnp.int32, (1, _LANE), 1) * _BM
    padded_start = jnp.int32(0)
    for e in range(E):
        m = (ids == e).astype(jnp.int32)
        csum = _cumsum2d(m)
        count = csum[_SUB - 1, _LANE - 1]
        dest = dest + m * (padded_start + csum - 1)
        padded_start = padded_start + ((count + _BM - 1) // _BM) * _BM
        if e < E - 1:
            bexp = bexp + (blk_iota >= padded_start).astype(jnp.int32)
    bexp = jnp.minimum(bexp, E - 1)
    lane = jax.lax.broadcasted_iota(jnp.int32, (1, _LANE), 1)
    first = (bexp != _masked_shift(bexp, 1, 1)).astype(jnp.int32)
    first = jnp.where(lane == 0, 1, first)
    runidx = first
    s = 1
    while s < _LANE:
        runidx = runidx + _masked_shift(runidx, s, 1)
        s *= 2
    runidx = runidx - 1
    dest_ref[...] = dest
    bexp_ref[...] = bexp
    first_ref[...] = first
    slot_ref[...] = runidx % _NBUF
    valid_ref[...] = (blk_iota < padded_start).astype(jnp.int32)


def _gemm_silu_kernel(
    be_ref, first_ref, slot_ref, valid_ref, x_ref, w_hbm, o_ref, wbuf, sems
):
    i = pl.program_id(0)
    nb = pl.num_programs(0)

    def _issue(j):
        @pl.when(jnp.logical_and(j < nb, first_ref[j] == 1))
        def _():
            for h in range(2):
                pltpu.make_async_copy(
                    w_hbm.at[be_ref[j], h],
                    wbuf.at[slot_ref[j], h],
                    sems.at[slot_ref[j], h],
                ).start()

    @pl.when(i == 0)
    def _():
        for j in range(_LOOKAHEAD):
            _issue(jnp.int32(j))

    _issue(i + _LOOKAHEAD)

    @pl.when(first_ref[i] == 1)
    def _():
        for h in range(2):
            pltpu.make_async_copy(
                w_hbm.at[be_ref[i], h],
                wbuf.at[slot_ref[i], h],
                sems.at[slot_ref[i], h],
            ).wait()

    @pl.when(valid_ref[i] == 1)
    def _():
        s = slot_ref[i]
        a = x_ref[...]
        g = jax.lax.dot_general(
            a, wbuf[s, 0], (((1,), (1,)), ((), ())), preferred_element_type=jnp.float32
        )
        u = jax.lax.dot_general(
            a, wbuf[s, 1], (((1,), (1,)), ((), ())), preferred_element_type=jnp.float32
        )
        o_ref[...] = g * jax.nn.sigmoid(g) * u


def kernel(local_hidden_states, up_weight, full_topk_ids):
    T, K = local_hidden_states.shape
    E, N, _ = up_weight.shape
    topk = full_topk_ids.shape[1]
    M = T * topk
    N2 = N // 2

    ids2d = full_topk_ids.reshape(_SUB, _LANE).astype(jnp.int32)

    M_pad = M + E * _BM
    num_blocks = M_pad // _BM

    dest2d, bexp, first, slot, valid = pl.pallas_call(
        functools.partial(_routing_kernel, E=E),
        out_shape=(
            jax.ShapeDtypeStruct((_SUB, _LANE), jnp.int32),
            jax.ShapeDtypeStruct((1, _LANE), jnp.int32),
            jax.ShapeDtypeStruct((1, _LANE), jnp.int32),
            jax.ShapeDtypeStruct((1, _LANE), jnp.int32),
            jax.ShapeDtypeStruct((1, _LANE), jnp.int32),
        ),
    )(ids2d)
    dest = dest2d.reshape(M)

    dest_t = dest.reshape(T, topk).T
    info = plsc.get_sparse_core_info()
    nw = info.num_cores * info.num_subcores
    t_per_w = T // nw

    @functools.partial(
        pl.kernel,
        mesh=plsc.VectorSubcoreMesh(core_axis_name="c", subcore_axis_name="s"),
        out_type=jax.ShapeDtypeStruct((M_pad, K), jnp.float32),
        scratch_types=[
            pltpu.VMEM((t_per_w,), jnp.int32),
            pltpu.VMEM((t_per_w,), jnp.int32),
            pltpu.VMEM((t_per_w, K), jnp.float32),
            pltpu.SemaphoreType.DMA,
            pltpu.SemaphoreType.DMA,
        ],
    )
    def _dispatch(x_hbm, dest_hbm, out_hbm, idx0, idx1, xv, sem0, sem1):
        wid = lax.axis_index("s") * info.num_cores + lax.axis_index("c")
        base = wid * t_per_w
        pltpu.sync_copy(dest_hbm.at[0, pl.ds(base, t_per_w)], idx0)
        pltpu.sync_copy(dest_hbm.at[1, pl.ds(base, t_per_w)], idx1)
        pltpu.sync_copy(x_hbm.at[pl.ds(base, t_per_w)], xv)
        c0 = pltpu.async_copy(xv, out_hbm.at[idx0], sem0)
        c1 = pltpu.async_copy(xv, out_hbm.at[idx1], sem1)
        c0.wait()
        c1.wait()

    x_sorted = _dispatch(local_hidden_states, dest_t)

    w4 = up_weight.reshape(E, 2, N2, K)

    grid_spec = pltpu.PrefetchScalarGridSpec(
        num_scalar_prefetch=4,
        grid=(num_blocks,),
        in_specs=[
            pl.BlockSpec((_BM, K), lambda i, be, fi, sl, va: (i, 0)),
            pl.BlockSpec(memory_space=pl.ANY),
        ],
        out_specs=pl.BlockSpec((_BM, N2), lambda i, be, fi, sl, va: (i, 0)),
        scratch_shapes=[
            pltpu.VMEM((_NBUF, 2, N2, K), jnp.float32),
            pltpu.SemaphoreType.DMA((_NBUF, 2)),
        ],
    )
    out_sorted = pl.pallas_call(
        _gemm_silu_kernel,
        grid_spec=grid_spec,
        out_shape=jax.ShapeDtypeStruct((M_pad, N2), jnp.float32),
        compiler_params=pltpu.CompilerParams(
            dimension_semantics=("arbitrary",),
        ),
    )(bexp[0], first[0], slot[0], valid[0], x_sorted, w4)

    return out_sorted[dest]


# speedup vs baseline: 1.3584x; 1.0444x over previous
import functools

import jax
import jax.numpy as jnp
from jax import lax
from jax.experimental import pallas as pl
from jax.experimental.pallas import tpu as pltpu
from jax.experimental.pallas import tpu_sc as plsc

_BM = 256
_SUB = 32
_LANE = 128


def _masked_shift(v, s, axis):
    rolled = jnp.roll(v, s, axis=axis)
    idx = jax.lax.broadcasted_iota(jnp.int32, v.shape, axis)
    return jnp.where(idx >= s, rolled, 0)


def _cumsum2d(m):
    s = 1
    while s < _LANE:
        m = m + _masked_shift(m, s, 1)
        s *= 2
    row_tot = jax.lax.broadcast_in_dim(m[:, _LANE - 1], (_SUB, 1), (0,))
    row_tot = jnp.broadcast_to(row_tot, (_SUB, _LANE))
    carry = _masked_shift(row_tot, 1, 0)
    s = 1
    while s < _SUB:
        carry = carry + _masked_shift(carry, s, 0)
        s *= 2
    return m + carry


_NBUF = 6
_LOOKAHEAD = 5


def _routing_kernel(ids_ref, dest_ref, bexp_ref, first_ref, slot_ref, valid_ref, E: int):
    ids = ids_ref[...]
    dest = jnp.zeros((_SUB, _LANE), jnp.int32)
    bexp = jnp.zeros((1, _LANE), jnp.int32)
    blk_iota = jax.lax.broadcasted_iota(jnp.int32, (1, _LANE), 1) * _BM
    padded_start = jnp.int32(0)
    for e in range(E):
        m = (ids == e).astype(jnp.int32)
        csum = _cumsum2d(m)
        count = csum[_SUB - 1, _LANE - 1]
        dest = dest + m * (padded_start + csum - 1)
        padded_start = padded_start + ((count + _BM - 1) // _BM) * _BM
        if e < E - 1:
            bexp = bexp + (blk_iota >= padded_start).astype(jnp.int32)
    bexp = jnp.minimum(bexp, E - 1)
    lane = jax.lax.broadcasted_iota(jnp.int32, (1, _LANE), 1)
    first = (bexp != _masked_shift(bexp, 1, 1)).astype(jnp.int32)
    first = jnp.where(lane == 0, 1, first)
    runidx = first
    s = 1
    while s < _LANE:
        runidx = runidx + _masked_shift(runidx, s, 1)
        s *= 2
    runidx = runidx - 1
    dest_ref[...] = dest
    bexp_ref[...] = bexp
    first_ref[...] = first
    slot_ref[...] = runidx % _NBUF
    valid_ref[...] = (blk_iota < padded_start).astype(jnp.int32)


def _gemm_silu_kernel(
    be_ref, first_ref, slot_ref, valid_ref, x_ref, w_hbm, o_ref, wbuf, sems
):
    i = pl.program_id(0)
    nb = pl.num_programs(0)

    def _issue(j):
        @pl.when(jnp.logical_and(j < nb, first_ref[j] == 1))
        def _():
            for h in range(2):
                pltpu.make_async_copy(
                    w_hbm.at[be_ref[j], h],
                    wbuf.at[slot_ref[j], h],
                    sems.at[slot_ref[j], h],
                ).start()

    @pl.when(i == 0)
    def _():
        for j in range(_LOOKAHEAD):
            _issue(jnp.int32(j))

    _issue(i + _LOOKAHEAD)

    @pl.when(first_ref[i] == 1)
    def _():
        for h in range(2):
            pltpu.make_async_copy(
                w_hbm.at[be_ref[i], h],
                wbuf.at[slot_ref[i], h],
                sems.at[slot_ref[i], h],
            ).wait()

    @pl.when(valid_ref[i] == 1)
    def _():
        s = slot_ref[i]
        a = x_ref[...]
        g = jax.lax.dot_general(
            a, wbuf[s, 0], (((1,), (1,)), ((), ())), preferred_element_type=jnp.float32
        )
        u = jax.lax.dot_general(
            a, wbuf[s, 1], (((1,), (1,)), ((), ())), preferred_element_type=jnp.float32
        )
        o_ref[...] = g * jax.nn.sigmoid(g) * u


def kernel(local_hidden_states, up_weight, full_topk_ids):
    T, K = local_hidden_states.shape
    E, N, _ = up_weight.shape
    topk = full_topk_ids.shape[1]
    M = T * topk
    N2 = N // 2

    ids2d = full_topk_ids.T.reshape(_SUB, _LANE).astype(jnp.int32)

    M_pad = M + E * _BM
    num_blocks = M_pad // _BM

    dest2d, bexp, first, slot, valid = pl.pallas_call(
        functools.partial(_routing_kernel, E=E),
        out_shape=(
            jax.ShapeDtypeStruct((_SUB, _LANE), jnp.int32),
            jax.ShapeDtypeStruct((1, _LANE), jnp.int32),
            jax.ShapeDtypeStruct((1, _LANE), jnp.int32),
            jax.ShapeDtypeStruct((1, _LANE), jnp.int32),
            jax.ShapeDtypeStruct((1, _LANE), jnp.int32),
        ),
    )(ids2d)

    dest_t = dest2d.reshape(topk, T)
    info = plsc.get_sparse_core_info()
    nw = info.num_cores * info.num_subcores
    t_per_w = T // nw

    @functools.partial(
        pl.kernel,
        mesh=plsc.VectorSubcoreMesh(core_axis_name="c", subcore_axis_name="s"),
        out_type=jax.ShapeDtypeStruct((M_pad, K), jnp.float32),
        scratch_types=[
            pltpu.VMEM((t_per_w,), jnp.int32),
            pltpu.VMEM((t_per_w,), jnp.int32),
            pltpu.VMEM((t_per_w, K), jnp.float32),
            pltpu.SemaphoreType.DMA,
            pltpu.SemaphoreType.DMA,
        ],
    )
    def _dispatch(x_hbm, dest_hbm, out_hbm, idx0, idx1, xv, sem0, sem1):
        wid = lax.axis_index("s") * info.num_cores + lax.axis_index("c")
        base = wid * t_per_w
        pltpu.sync_copy(dest_hbm.at[0, pl.ds(base, t_per_w)], idx0)
        pltpu.sync_copy(dest_hbm.at[1, pl.ds(base, t_per_w)], idx1)
        pltpu.sync_copy(x_hbm.at[pl.ds(base, t_per_w)], xv)
        c0 = pltpu.async_copy(xv, out_hbm.at[idx0], sem0)
        c1 = pltpu.async_copy(xv, out_hbm.at[idx1], sem1)
        c0.wait()
        c1.wait()

    x_sorted = _dispatch(local_hidden_states, dest_t)

    w4 = up_weight.reshape(E, 2, N2, K)

    grid_spec = pltpu.PrefetchScalarGridSpec(
        num_scalar_prefetch=4,
        grid=(num_blocks,),
        in_specs=[
            pl.BlockSpec((_BM, K), lambda i, be, fi, sl, va: (i * va[i], 0)),
            pl.BlockSpec(memory_space=pl.ANY),
        ],
        out_specs=pl.BlockSpec((_BM, N2), lambda i, be, fi, sl, va: (i, 0)),
        scratch_shapes=[
            pltpu.VMEM((_NBUF, 2, N2, K), jnp.float32),
            pltpu.SemaphoreType.DMA((_NBUF, 2)),
        ],
    )
    out_sorted = pl.pallas_call(
        _gemm_silu_kernel,
        grid_spec=grid_spec,
        out_shape=jax.ShapeDtypeStruct((M_pad, N2), jnp.float32),
        compiler_params=pltpu.CompilerParams(
            dimension_semantics=("arbitrary",),
        ),
    )(bexp[0], first[0], slot[0], valid[0], x_sorted, w4)

    dest_p = dest_t.T.reshape(M)
    return out_sorted[dest_p]


# speedup vs baseline: 1.3766x; 1.0134x over previous
import functools

import jax
import jax.numpy as jnp
from jax import lax
from jax.experimental import pallas as pl
from jax.experimental.pallas import tpu as pltpu
from jax.experimental.pallas import tpu_sc as plsc

_BM = 256
_SUB = 32
_LANE = 128


def _masked_shift(v, s, axis):
    rolled = jnp.roll(v, s, axis=axis)
    idx = jax.lax.broadcasted_iota(jnp.int32, v.shape, axis)
    return jnp.where(idx >= s, rolled, 0)


def _cumsum2d(m):
    s = 1
    while s < _LANE:
        m = m + _masked_shift(m, s, 1)
        s *= 2
    row_tot = jax.lax.broadcast_in_dim(m[:, _LANE - 1], (_SUB, 1), (0,))
    row_tot = jnp.broadcast_to(row_tot, (_SUB, _LANE))
    carry = _masked_shift(row_tot, 1, 0)
    s = 1
    while s < _SUB:
        carry = carry + _masked_shift(carry, s, 0)
        s *= 2
    return m + carry


_NBUF = 6
_LOOKAHEAD = 5


def _routing_kernel(ids_ref, dest_ref, bexp_ref, first_ref, slot_ref, valid_ref, E: int):
    ids = ids_ref[...]
    dest = jnp.zeros((_SUB, _LANE), jnp.int32)
    bexp = jnp.zeros((1, _LANE), jnp.int32)
    blk_iota = jax.lax.broadcasted_iota(jnp.int32, (1, _LANE), 1) * _BM
    padded_start = jnp.int32(0)
    for e in range(E):
        m = (ids == e).astype(jnp.int32)
        csum = _cumsum2d(m)
        count = csum[_SUB - 1, _LANE - 1]
        dest = dest + m * (padded_start + csum - 1)
        padded_start = padded_start + ((count + _BM - 1) // _BM) * _BM
        if e < E - 1:
            bexp = bexp + (blk_iota >= padded_start).astype(jnp.int32)
    bexp = jnp.minimum(bexp, E - 1)
    lane = jax.lax.broadcasted_iota(jnp.int32, (1, _LANE), 1)
    first = (bexp != _masked_shift(bexp, 1, 1)).astype(jnp.int32)
    first = jnp.where(lane == 0, 1, first)
    runidx = first
    s = 1
    while s < _LANE:
        runidx = runidx + _masked_shift(runidx, s, 1)
        s *= 2
    runidx = runidx - 1
    dest_ref[...] = dest
    bexp_ref[...] = bexp
    first_ref[...] = first
    slot_ref[...] = runidx % _NBUF
    valid_ref[...] = (blk_iota < padded_start).astype(jnp.int32)


def _gemm_silu_kernel(
    be_ref, first_ref, slot_ref, valid_ref, x_ref, w_hbm, o_ref, wbuf, sems
):
    i = pl.program_id(0)
    nb = pl.num_programs(0)

    def _issue(j):
        @pl.when(jnp.logical_and(j < nb, first_ref[j] == 1))
        def _():
            for h in range(2):
                pltpu.make_async_copy(
                    w_hbm.at[be_ref[j], h],
                    wbuf.at[slot_ref[j], h],
                    sems.at[slot_ref[j], h],
                ).start()

    @pl.when(i == 0)
    def _():
        for j in range(_LOOKAHEAD):
            _issue(jnp.int32(j))

    _issue(i + _LOOKAHEAD)

    @pl.when(first_ref[i] == 1)
    def _():
        for h in range(2):
            pltpu.make_async_copy(
                w_hbm.at[be_ref[i], h],
                wbuf.at[slot_ref[i], h],
                sems.at[slot_ref[i], h],
            ).wait()

    @pl.when(valid_ref[i] == 1)
    def _():
        s = slot_ref[i]
        a = x_ref[...]
        g = jax.lax.dot_general(
            a, wbuf[s, 0], (((1,), (1,)), ((), ())), preferred_element_type=jnp.float32
        )
        u = jax.lax.dot_general(
            a, wbuf[s, 1], (((1,), (1,)), ((), ())), preferred_element_type=jnp.float32
        )
        o_ref[...] = g * jax.nn.sigmoid(g) * u


def kernel(local_hidden_states, up_weight, full_topk_ids):
    T, K = local_hidden_states.shape
    E, N, _ = up_weight.shape
    topk = full_topk_ids.shape[1]
    M = T * topk
    N2 = N // 2

    ids2d = full_topk_ids.T.reshape(_SUB, _LANE).astype(jnp.int32)

    M_pad = M + E * _BM
    num_blocks = M_pad // _BM

    dest2d, bexp, first, slot, valid = pl.pallas_call(
        functools.partial(_routing_kernel, E=E),
        out_shape=(
            jax.ShapeDtypeStruct((_SUB, _LANE), jnp.int32),
            jax.ShapeDtypeStruct((1, _LANE), jnp.int32),
            jax.ShapeDtypeStruct((1, _LANE), jnp.int32),
            jax.ShapeDtypeStruct((1, _LANE), jnp.int32),
            jax.ShapeDtypeStruct((1, _LANE), jnp.int32),
        ),
    )(ids2d)

    dest_t = dest2d.reshape(topk, T)
    info = plsc.get_sparse_core_info()
    nw = info.num_cores * info.num_subcores
    t_per_w = T // nw

    @functools.partial(
        pl.kernel,
        mesh=plsc.VectorSubcoreMesh(core_axis_name="c", subcore_axis_name="s"),
        out_type=jax.ShapeDtypeStruct((M_pad, K), jnp.float32),
        scratch_types=[
            pltpu.VMEM((t_per_w,), jnp.int32),
            pltpu.VMEM((t_per_w,), jnp.int32),
            pltpu.VMEM((t_per_w, K), jnp.float32),
            pltpu.SemaphoreType.DMA,
            pltpu.SemaphoreType.DMA,
        ],
    )
    def _dispatch(x_hbm, dest_hbm, out_hbm, idx0, idx1, xv, sem0, sem1):
        wid = lax.axis_index("s") * info.num_cores + lax.axis_index("c")
        base = wid * t_per_w
        pltpu.sync_copy(dest_hbm.at[0, pl.ds(base, t_per_w)], idx0)
        pltpu.sync_copy(dest_hbm.at[1, pl.ds(base, t_per_w)], idx1)
        pltpu.sync_copy(x_hbm.at[pl.ds(base, t_per_w)], xv)
        c0 = pltpu.async_copy(xv, out_hbm.at[idx0], sem0)
        c1 = pltpu.async_copy(xv, out_hbm.at[idx1], sem1)
        c0.wait()
        c1.wait()

    x_sorted = _dispatch(local_hidden_states, dest_t)

    w4 = up_weight.reshape(E, 2, N2, K)

    grid_spec = pltpu.PrefetchScalarGridSpec(
        num_scalar_prefetch=4,
        grid=(num_blocks,),
        in_specs=[
            pl.BlockSpec((_BM, K), lambda i, be, fi, sl, va: (i * va[i], 0)),
            pl.BlockSpec(memory_space=pl.ANY),
        ],
        out_specs=pl.BlockSpec((_BM, N2), lambda i, be, fi, sl, va: (i, 0)),
        scratch_shapes=[
            pltpu.VMEM((_NBUF, 2, N2, K), jnp.float32),
            pltpu.SemaphoreType.DMA((_NBUF, 2)),
        ],
    )
    out_sorted = pl.pallas_call(
        _gemm_silu_kernel,
        grid_spec=grid_spec,
        out_shape=jax.ShapeDtypeStruct((M_pad, N2), jnp.float32),
        compiler_params=pltpu.CompilerParams(
            dimension_semantics=("arbitrary",),
        ),
    )(bexp[0], first[0], slot[0], valid[0], x_sorted, w4)

    dest_p = dest_t.T.reshape(M)
    p_per_w = M // nw
    _CH = 32
    n_ch = p_per_w // _CH

    @functools.partial(
        pl.kernel,
        mesh=plsc.VectorSubcoreMesh(core_axis_name="c", subcore_axis_name="s"),
        out_type=jax.ShapeDtypeStruct((M, N2), jnp.float32),
        scratch_types=[
            pltpu.VMEM((p_per_w,), jnp.int32),
            pltpu.VMEM((_CH, N2), jnp.float32),
            pltpu.VMEM((_CH, N2), jnp.float32),
            pltpu.SemaphoreType.DMA,
            pltpu.SemaphoreType.DMA,
        ],
    )
    def _unperm(src_hbm, destp_hbm, out_hbm, idx_v, buf0, buf1, sem0, sem1):
        wid = lax.axis_index("s") * info.num_cores + lax.axis_index("c")
        base = wid * p_per_w
        pltpu.sync_copy(destp_hbm.at[pl.ds(base, p_per_w)], idx_v)
        bufs = (buf0, buf1)
        sems = (sem0, sem1)
        copies = [None, None]
        copies[0] = pltpu.async_copy(
            src_hbm.at[idx_v.at[pl.ds(0, _CH)]], buf0, sem0
        )
        for c in range(n_ch):
            nxt = c + 1
            if nxt < n_ch:
                copies[nxt % 2] = pltpu.async_copy(
                    src_hbm.at[idx_v.at[pl.ds(nxt * _CH, _CH)]],
                    bufs[nxt % 2],
                    sems[nxt % 2],
                )
            copies[c % 2].wait()
            pltpu.sync_copy(bufs[c % 2], out_hbm.at[pl.ds(base + c * _CH, _CH)])

    return _unperm(out_sorted, dest_p)
